# baseline (speedup 1.0000x reference)
import functools

import jax
import jax.numpy as jnp
from jax import lax
from jax.experimental import pallas as pl
from jax.experimental.pallas import tpu as pltpu
from jax.experimental.pallas import tpu_sc as plsc

_TM = 1024


def _argmax_body(x_ref, cb_ref, idx_ref):
    xb = x_ref[...]
    norm = jnp.sqrt(jnp.sum(xb * xb, axis=1, keepdims=True))
    emb = xb / jnp.maximum(norm, 1e-12)
    sims = lax.dot_general(
        emb,
        cb_ref[...],
        dimension_numbers=(((1,), (1,)), ((), ())),
        preferred_element_type=jnp.float32,
    )
    tm, v = sims.shape
    nc = v // 128
    cols = [sims[:, c * 128:(c + 1) * 128] for c in range(nc)]
    while len(cols) > 1:
        cols = [
            jnp.maximum(cols[2 * i], cols[2 * i + 1])
            for i in range(len(cols) // 2)
        ]
    rowmax = jnp.max(cols[0], axis=1, keepdims=True)
    acc = jnp.full((tm, 128), nc, dtype=jnp.int32)
    for c in range(nc - 1, -1, -1):
        acc = jnp.where(
            sims[:, c * 128:(c + 1) * 128] == rowmax,
            jnp.int32(c),
            acc,
        )
    lane = jax.lax.broadcasted_iota(jnp.int32, (tm, 128), 1)
    jcand = jnp.where(acc < nc, acc * 128 + lane, jnp.int32(v))
    idx = jnp.min(jcand, axis=1)
    idx_ref[...] = idx.reshape(idx_ref.shape)


def _best_indices(x, codebook):
    n, d = x.shape
    v = codebook.shape[0]
    grid = n // _TM
    out = pl.pallas_call(
        _argmax_body,
        grid=(grid,),
        in_specs=[
            pl.BlockSpec((_TM, d), lambda i: (i, 0)),
            pl.BlockSpec((v, d), lambda i: (0, 0)),
        ],
        out_specs=pl.BlockSpec((_TM, 1), lambda i: (i, 0)),
        out_shape=jax.ShapeDtypeStruct((n, 1), jnp.int32),
        compiler_params=pltpu.CompilerParams(
            dimension_semantics=("parallel",)
        ),
    )(x, codebook)
    return out.reshape(n)


@functools.cache
def _make_gather(v, d, b):
    info = plsc.get_sparse_core_info()
    nw = info.num_cores * info.num_subcores
    b_per_w = b // nw
    chunk = 128
    n_chunks = b_per_w // chunk
    mesh = plsc.VectorSubcoreMesh(core_axis_name="c", subcore_axis_name="s")

    @functools.partial(
        pl.kernel,
        mesh=mesh,
        out_type=jax.ShapeDtypeStruct((b, d), jnp.float32),
        scratch_types=[
            pltpu.VMEM((b_per_w,), jnp.int32),
            pltpu.VMEM((b_per_w, d), jnp.float32),
            pltpu.SemaphoreType.DMA,
        ],
        compiler_params=pltpu.CompilerParams(use_tc_tiling_on_sc=False),
    )
    def gather(table_hbm, idx_hbm, out_hbm, idx_v, rows_v, sem):
        wid = lax.axis_index("s") * info.num_cores + lax.axis_index("c")
        base = wid * b_per_w
        pltpu.sync_copy(idx_hbm.at[pl.ds(base, b_per_w)], idx_v)
        copies = [
            pltpu.async_copy(
                table_hbm.at[idx_v.at[pl.ds(c * chunk, chunk)]],
                rows_v.at[pl.ds(c * chunk, chunk)],
                sem,
            )
            for c in range(n_chunks)
        ]
        for cp in copies:
            cp.wait()
        pltpu.sync_copy(rows_v, out_hbm.at[pl.ds(base, b_per_w)])

    return gather


def kernel(x, codebook):
    idx = _best_indices(x, codebook)
    return _make_gather(codebook.shape[0], codebook.shape[1], x.shape[0])(
        codebook, idx
    )


# speedup vs baseline: 1.3275x; 1.3275x over previous
import functools

import jax
import jax.numpy as jnp
from jax import lax
from jax.experimental import pallas as pl
from jax.experimental.pallas import tpu as pltpu
from jax.experimental.pallas import tpu_sc as plsc

_TM = 1024


def _argmax_body(x_ref, cb_ref, idx_ref):
    xb = x_ref[...]
    norm = jnp.sqrt(jnp.sum(xb * xb, axis=1, keepdims=True))
    emb = xb / jnp.maximum(norm, 1e-12)
    sims = lax.dot_general(
        emb,
        cb_ref[...],
        dimension_numbers=(((1,), (1,)), ((), ())),
        preferred_element_type=jnp.float32,
    )
    idx = jnp.argmax(sims, axis=1).astype(jnp.int32)
    idx_ref[...] = idx.reshape(idx_ref.shape)


def _best_indices(x, codebook):
    n, d = x.shape
    v = codebook.shape[0]
    grid = n // _TM
    out = pl.pallas_call(
        _argmax_body,
        grid=(grid,),
        in_specs=[
            pl.BlockSpec((_TM, d), lambda i: (i, 0)),
            pl.BlockSpec((v, d), lambda i: (0, 0)),
        ],
        out_specs=pl.BlockSpec((_TM // 128, 128), lambda i: (i, 0)),
        out_shape=jax.ShapeDtypeStruct((n // 128, 128), jnp.int32),
        compiler_params=pltpu.CompilerParams(
            dimension_semantics=("parallel",)
        ),
    )(x, codebook)
    return out.reshape(n)


@functools.cache
def _make_gather(v, d, b):
    info = plsc.get_sparse_core_info()
    nw = info.num_cores * info.num_subcores
    b_per_w = b // nw
    chunk = 128
    n_chunks = b_per_w // chunk
    mesh = plsc.VectorSubcoreMesh(core_axis_name="c", subcore_axis_name="s")

    @functools.partial(
        pl.kernel,
        mesh=mesh,
        out_type=jax.ShapeDtypeStruct((b, d), jnp.float32),
        scratch_types=[
            pltpu.VMEM((b_per_w,), jnp.int32),
            pltpu.VMEM((b_per_w, d), jnp.float32),
            pltpu.SemaphoreType.DMA,
        ],
        compiler_params=pltpu.CompilerParams(use_tc_tiling_on_sc=False),
    )
    def gather(table_hbm, idx_hbm, out_hbm, idx_v, rows_v, sem):
        wid = lax.axis_index("s") * info.num_cores + lax.axis_index("c")
        base = wid * b_per_w
        pltpu.sync_copy(idx_hbm.at[pl.ds(base, b_per_w)], idx_v)
        copies = [
            pltpu.async_copy(
                table_hbm.at[idx_v.at[pl.ds(c * chunk, chunk)]],
                rows_v.at[pl.ds(c * chunk, chunk)],
                sem,
            )
            for c in range(n_chunks)
        ]
        for cp in copies:
            cp.wait()
        pltpu.sync_copy(rows_v, out_hbm.at[pl.ds(base, b_per_w)])

    return gather


def kernel(x, codebook):
    idx = _best_indices(x, codebook)
    return _make_gather(codebook.shape[0], codebook.shape[1], x.shape[0])(
        codebook, idx
    )
